# baseline (speedup 1.0000x reference)
import functools

import jax
import jax.numpy as jnp
from jax import lax
from jax.experimental import pallas as pl
from jax.experimental.pallas import tpu as pltpu
from jax.experimental.pallas import tpu_sc as plsc

_N = 10000
_E = 320000
_D = 128
_L = 6

_NT = 16
_CHUNK = 128
_NP = 10240
_ROWS_PER_TILE = _NP // _NT
_EP = 327680
_CH = _EP // (_NT * _CHUNK)
_G = 16


def _sc_body(w_hbm, wp_hbm, src_hbm, dst_hbm, den_hbm, num_hbm,
             sidx, didx, rows, acc, sem0, sem1):
    cid = lax.axis_index("c")
    tid = lax.axis_index("s")
    sems = (sem0, sem1)

    def run_core(tab_hbm, out_hbm):
        zv = jnp.zeros((16,), jnp.float32)

        def zrow(i, carry):
            for c in range(8):
                rows[0, i, pl.ds(c * 16, 16)] = zv
            return carry

        lax.fori_loop(0, 128, zrow, 0)
        base = tid * _ROWS_PER_TILE
        for k in range(_ROWS_PER_TILE // 128):
            pltpu.sync_copy(rows.at[0], acc.at[pl.ds(base + k * 128, 128)])
        plsc.subcore_barrier()

        def group(g, carry):
            gbase = g * _G
            pltpu.sync_copy(src_hbm.at[tid, pl.ds(gbase, _G)], sidx)
            pltpu.sync_copy(dst_hbm.at[tid, pl.ds(gbase, _G)], didx)
            for b in range(2):
                pltpu.async_copy(tab_hbm.at[sidx.at[b]], rows.at[b], sems[b])

            def chunk(i, carry2):
                j = i * 2
                for b in range(2):
                    jj = j + b
                    pltpu.make_async_copy(
                        tab_hbm.at[sidx.at[jj]], rows.at[b], sems[b]).wait()
                    pltpu.sync_copy(rows.at[b], acc.at[didx.at[jj]], add=True)

                    @pl.when(jj + 2 < _G)
                    def _():
                        pltpu.async_copy(
                            tab_hbm.at[sidx.at[jj + 2]], rows.at[b], sems[b])
                return carry2

            lax.fori_loop(0, _G // 2, chunk, 0)
            return carry

        lax.fori_loop(0, _CH // _G, group, 0)
        plsc.subcore_barrier()

        pltpu.sync_copy(acc.at[pl.ds(base, _ROWS_PER_TILE)],
                        out_hbm.at[pl.ds(base, _ROWS_PER_TILE)])

    @pl.when(cid == 0)
    def _():
        run_core(w_hbm, den_hbm)

    @pl.when(cid == 1)
    def _():
        run_core(wp_hbm, num_hbm)


def _sc_segment_sums(w_tab, wp_tab, src3, dst3):
    mesh = plsc.VectorSubcoreMesh(core_axis_name="c", subcore_axis_name="s")
    fn = functools.partial(
        pl.kernel,
        mesh=mesh,
        out_type=[
            jax.ShapeDtypeStruct((_NP, _D), jnp.float32),
            jax.ShapeDtypeStruct((_NP, _D), jnp.float32),
        ],
        scratch_types=[
            pltpu.VMEM((_G, _CHUNK), jnp.int32),
            pltpu.VMEM((_G, _CHUNK), jnp.int32),
            pltpu.VMEM((2, _CHUNK, _D), jnp.float32),
            pltpu.VMEM_SHARED((_NP, _D), jnp.float32),
            pltpu.SemaphoreType.DMA,
            pltpu.SemaphoreType.DMA,
        ],
    )(_sc_body)
    return fn(w_tab, wp_tab, src3, dst3)


_NW = 2 * _NT
_WE = _EP // _NW
_B = 80


def _move_body(combo_hbm, pos_hbm, out_hbm, cin, pin, cout, v16):
    cid = lax.axis_index("c")
    tid = lax.axis_index("s")
    pltpu.sync_copy(combo_hbm.at[cid, tid], cin)
    pltpu.sync_copy(pos_hbm.at[cid, tid], pin)

    def step(i, carry):
        sl = pl.ds(i * 16, 16)
        plsc.store_scatter(cout, [pin[sl]], cin[sl])
        return carry

    lax.fori_loop(0, _WE // 16, step, 0)
    pltpu.sync_copy(cout, out_hbm.at[cid, tid])


def _sc_partition(combo, pos):
    mesh = plsc.VectorSubcoreMesh(core_axis_name="c", subcore_axis_name="s")
    fn = functools.partial(
        pl.kernel,
        mesh=mesh,
        compiler_params=pltpu.CompilerParams(needs_layout_passes=False),
        out_type=jax.ShapeDtypeStruct((2, _NT, _WE), jnp.int32),
        scratch_types=[
            pltpu.VMEM((_WE,), jnp.int32),
            pltpu.VMEM((_WE,), jnp.int32),
            pltpu.VMEM((_WE,), jnp.int32),
            pltpu.VMEM((16,), jnp.int32),
        ],
    )(_move_body)
    return fn(combo, pos)


def _partition_edges(src_p, dst_p):
    combo = (src_p * 32768 + dst_p).reshape(2, _NT, _WE)
    b = (src_p >> 7).reshape(2, _NT, _WE)
    onehot = (b[..., None] == jnp.arange(_B, dtype=jnp.int32)).astype(jnp.int32)
    run = jnp.cumsum(onehot, axis=2)
    cnt = run[:, :, -1, :]
    off = jnp.cumsum(cnt, axis=-1) - cnt
    rank = jnp.sum(run * onehot, axis=-1) - 1
    pos = rank + jnp.take_along_axis(off, b, axis=-1)
    moved = _sc_partition(combo, pos.astype(jnp.int32))
    return moved >> 15, moved & 32767


_BR = 2048


def _ln_relu(h, g, b):
    mu = jnp.mean(h, axis=1, keepdims=True)
    var = jnp.mean((h - mu) ** 2, axis=1, keepdims=True)
    z = (h - mu) * lax.rsqrt(var + 1e-5) * g + b
    return jnp.maximum(z, 0.0)


def _tables(z):
    p = z + 1e-7
    w = jnp.exp(p)
    return w, w * p


def _tc_pre_body(x_ref, g_ref, b_ref, w_ref, wp_ref):
    z = _ln_relu(x_ref[...], g_ref[...], b_ref[...])
    w, wp = _tables(z)
    w_ref[...] = w
    wp_ref[...] = wp


def _tc_mid_body(h_ref, num_ref, den_ref, W_ref, b_ref, g_ref, lb_ref,
                 g2_ref, lb2_ref, h2_ref, w_ref, wp_ref):
    h = h_ref[...]
    z = _ln_relu(h, g_ref[...], lb_ref[...])
    aggr = num_ref[...] / jnp.maximum(den_ref[...], 1e-16)
    h2 = h + jnp.dot(z + aggr, W_ref[...],
                     preferred_element_type=jnp.float32) + b_ref[...]
    h2_ref[...] = h2
    z2 = _ln_relu(h2, g2_ref[...], lb2_ref[...])
    w, wp = _tables(z2)
    w_ref[...] = w
    wp_ref[...] = wp


def _tc_final_body(h_ref, num_ref, den_ref, W_ref, b_ref, g_ref, lb_ref,
                   linW_ref, linb_ref, hw_ref, hb_ref, mask_ref, out_ref):
    h = h_ref[...]
    z = _ln_relu(h, g_ref[...], lb_ref[...])
    aggr = num_ref[...] / jnp.maximum(den_ref[...], 1e-16)
    h2 = h + jnp.dot(z + aggr, W_ref[...],
                     preferred_element_type=jnp.float32) + b_ref[...]
    hr = jnp.maximum(
        jnp.dot(h2, linW_ref[...], preferred_element_type=jnp.float32)
        + linb_ref[...], 0.0)
    o = jnp.sum(hr * hw_ref[...], axis=1, keepdims=True) + hb_ref[...]
    out_ref[...] = jnp.where(mask_ref[...] > 0, o, 0.0)


def _row_spec():
    return pl.BlockSpec((_BR, _D), lambda i: (i, 0))


def _bcast_spec(shape):
    return pl.BlockSpec(shape, lambda i: tuple(0 for _ in shape))


def _tc_pre(xp, g, b):
    return pl.pallas_call(
        _tc_pre_body,
        grid=(_NP // _BR,),
        in_specs=[_row_spec(), _bcast_spec((1, _D)), _bcast_spec((1, _D))],
        out_specs=[_row_spec(), _row_spec()],
        out_shape=[jax.ShapeDtypeStruct((_NP, _D), jnp.float32)] * 2,
    )(xp, g, b)


def _tc_mid(h, num, den, W, b, g, lb, g2, lb2):
    return pl.pallas_call(
        _tc_mid_body,
        grid=(_NP // _BR,),
        in_specs=[_row_spec(), _row_spec(), _row_spec(),
                  _bcast_spec((_D, _D)), _bcast_spec((1, _D)),
                  _bcast_spec((1, _D)), _bcast_spec((1, _D)),
                  _bcast_spec((1, _D)), _bcast_spec((1, _D))],
        out_specs=[_row_spec(), _row_spec(), _row_spec()],
        out_shape=[jax.ShapeDtypeStruct((_NP, _D), jnp.float32)] * 3,
    )(h, num, den, W, b, g, lb, g2, lb2)


def _tc_final(h, num, den, W, b, g, lb, linW, linb, hw, hb, maskf):
    return pl.pallas_call(
        _tc_final_body,
        grid=(_NP // _BR,),
        in_specs=[_row_spec(), _row_spec(), _row_spec(),
                  _bcast_spec((_D, _D)), _bcast_spec((1, _D)),
                  _bcast_spec((1, _D)), _bcast_spec((1, _D)),
                  _bcast_spec((_D, _D)), _bcast_spec((1, _D)),
                  _bcast_spec((1, _D)), _bcast_spec((1, 1)),
                  pl.BlockSpec((_BR, 1), lambda i: (i, 0))],
        out_specs=pl.BlockSpec((_BR, 1), lambda i: (i, 0)),
        out_shape=jax.ShapeDtypeStruct((_NP, 1), jnp.float32),
    )(h, num, den, W, b, g, lb, linW, linb, hw, hb, maskf)


def kernel(x, edge_index, regression_mask, ln_g, ln_b, conv_W, conv_b,
           lin_W, lin_b, head_W, head_b):
    n, d = x.shape
    e = edge_index.shape[1]
    assert (n, d, e, conv_W.shape[0]) == (_N, _D, _E, _L)

    f32 = jnp.float32
    xp = jnp.concatenate([x, jnp.zeros((_NP - _N, _D), f32)], axis=0)
    maskf = jnp.concatenate(
        [regression_mask.astype(f32), jnp.zeros((_NP - _N,), f32)]
    ).reshape(_NP, 1)

    pad = jnp.full((_EP - _E,), _N, jnp.int32)
    src_p = jnp.concatenate([edge_index[0].astype(jnp.int32), pad])
    dst_p = jnp.concatenate([edge_index[1].astype(jnp.int32), pad])
    src_m, dst_m = _partition_edges(src_p, dst_p)
    src3 = src_m.reshape(_NT, _CH, _CHUNK)
    dst3 = dst_m.reshape(_NT, _CH, _CHUNK)

    def r1(v):
        return v.reshape(1, -1)

    h = xp
    w, wp = _tc_pre(xp, r1(ln_g[0]), r1(ln_b[0]))
    for l in range(_L):
        den, num = _sc_segment_sums(w, wp, src3, dst3)
        if l < _L - 1:
            h, w, wp = _tc_mid(h, num, den, conv_W[l], r1(conv_b[l]),
                               r1(ln_g[l]), r1(ln_b[l]),
                               r1(ln_g[l + 1]), r1(ln_b[l + 1]))
        else:
            out2 = _tc_final(h, num, den, conv_W[l], r1(conv_b[l]),
                             r1(ln_g[l]), r1(ln_b[l]),
                             lin_W, r1(lin_b), r1(head_W[:, 0]),
                             head_b.reshape(1, 1), maskf)
    return out2[:_N, 0]


# speedup vs baseline: 2.3858x; 2.3858x over previous
import functools

import jax
import jax.numpy as jnp
from jax import lax
from jax.experimental import pallas as pl
from jax.experimental.pallas import tpu as pltpu
from jax.experimental.pallas import tpu_sc as plsc

_N = 10000
_E = 320000
_D = 128
_L = 6

_NT = 16
_CHUNK = 128
_NP = 10240
_ROWS_PER_TILE = _NP // _NT
_EP = 327680
_CH = _EP // (_NT * _CHUNK)
_G = 16


def _sc_body(w_hbm, wp_hbm, src_hbm, dst_hbm, den_hbm, num_hbm,
             sidx, didx, rows, acc, *sems):
    cid = lax.axis_index("c")
    tid = lax.axis_index("s")

    def run_core(tab_hbm, out_hbm):
        zv = jnp.zeros((16,), jnp.float32)

        def zrow(i, carry):
            for c in range(8):
                rows[0, i, pl.ds(c * 16, 16)] = zv
            return carry

        lax.fori_loop(0, 128, zrow, 0)
        base = tid * _ROWS_PER_TILE
        for k in range(_ROWS_PER_TILE // 128):
            pltpu.sync_copy(rows.at[0], acc.at[pl.ds(base + k * 128, 128)])
        plsc.subcore_barrier()

        def group(g, carry):
            gbase = g * _G
            pltpu.sync_copy(src_hbm.at[tid, pl.ds(gbase, _G)], sidx)
            pltpu.sync_copy(dst_hbm.at[tid, pl.ds(gbase, _G)], didx)
            def start(jj, b):
                for q in range(4):
                    qs = pl.ds(q * 32, 32)
                    pltpu.async_copy(tab_hbm.at[sidx.at[jj, qs]],
                                     rows.at[b, qs], sems[4 * b + q])

            def drain(jj, b):
                for q in range(4):
                    qs = pl.ds(q * 32, 32)
                    pltpu.make_async_copy(tab_hbm.at[sidx.at[jj, qs]],
                                          rows.at[b, qs],
                                          sems[4 * b + q]).wait()

            for b in range(2):
                start(b, b)

            def chunk(i, carry2):
                j = i * 2
                for b in range(2):
                    jj = j + b
                    drain(jj, b)
                    pltpu.sync_copy(rows.at[b], acc.at[didx.at[jj]], add=True)

                    @pl.when(jj + 2 < _G)
                    def _():
                        start(jj + 2, b)
                return carry2

            lax.fori_loop(0, _G // 2, chunk, 0)
            return carry

        lax.fori_loop(0, _CH // _G, group, 0)
        plsc.subcore_barrier()

        pltpu.sync_copy(acc.at[pl.ds(base, _ROWS_PER_TILE)],
                        out_hbm.at[pl.ds(base, _ROWS_PER_TILE)])

    @pl.when(cid == 0)
    def _():
        run_core(w_hbm, den_hbm)

    @pl.when(cid == 1)
    def _():
        run_core(wp_hbm, num_hbm)


def _sc_segment_sums(w_tab, wp_tab, src3, dst3):
    mesh = plsc.VectorSubcoreMesh(core_axis_name="c", subcore_axis_name="s")
    fn = functools.partial(
        pl.kernel,
        mesh=mesh,
        out_type=[
            jax.ShapeDtypeStruct((_NP, _D), jnp.float32),
            jax.ShapeDtypeStruct((_NP, _D), jnp.float32),
        ],
        scratch_types=[
            pltpu.VMEM((_G, _CHUNK), jnp.int32),
            pltpu.VMEM((_G, _CHUNK), jnp.int32),
            pltpu.VMEM((2, _CHUNK, _D), jnp.float32),
            pltpu.VMEM_SHARED((_NP, _D), jnp.float32),
        ] + [pltpu.SemaphoreType.DMA] * 8,
    )(_sc_body)
    return fn(w_tab, wp_tab, src3, dst3)


_BR = 2048


def _ln_relu(h, g, b):
    mu = jnp.mean(h, axis=1, keepdims=True)
    var = jnp.mean((h - mu) ** 2, axis=1, keepdims=True)
    z = (h - mu) * lax.rsqrt(var + 1e-5) * g + b
    return jnp.maximum(z, 0.0)


def _tables(z):
    p = z + 1e-7
    w = jnp.exp(p)
    return w, w * p


def _tc_pre_body(x_ref, g_ref, b_ref, w_ref, wp_ref):
    z = _ln_relu(x_ref[...], g_ref[...], b_ref[...])
    w, wp = _tables(z)
    w_ref[...] = w
    wp_ref[...] = wp


def _tc_mid_body(h_ref, num_ref, den_ref, W_ref, b_ref, g_ref, lb_ref,
                 g2_ref, lb2_ref, h2_ref, w_ref, wp_ref):
    h = h_ref[...]
    z = _ln_relu(h, g_ref[...], lb_ref[...])
    aggr = num_ref[...] / jnp.maximum(den_ref[...], 1e-16)
    h2 = h + jnp.dot(z + aggr, W_ref[...],
                     preferred_element_type=jnp.float32) + b_ref[...]
    h2_ref[...] = h2
    z2 = _ln_relu(h2, g2_ref[...], lb2_ref[...])
    w, wp = _tables(z2)
    w_ref[...] = w
    wp_ref[...] = wp


def _tc_final_body(h_ref, num_ref, den_ref, W_ref, b_ref, g_ref, lb_ref,
                   linW_ref, linb_ref, hw_ref, hb_ref, mask_ref, out_ref):
    h = h_ref[...]
    z = _ln_relu(h, g_ref[...], lb_ref[...])
    aggr = num_ref[...] / jnp.maximum(den_ref[...], 1e-16)
    h2 = h + jnp.dot(z + aggr, W_ref[...],
                     preferred_element_type=jnp.float32) + b_ref[...]
    hr = jnp.maximum(
        jnp.dot(h2, linW_ref[...], preferred_element_type=jnp.float32)
        + linb_ref[...], 0.0)
    o = jnp.sum(hr * hw_ref[...], axis=1, keepdims=True) + hb_ref[...]
    out_ref[...] = jnp.where(mask_ref[...] > 0, o, 0.0)


def _row_spec():
    return pl.BlockSpec((_BR, _D), lambda i: (i, 0))


def _bcast_spec(shape):
    return pl.BlockSpec(shape, lambda i: tuple(0 for _ in shape))


def _tc_pre(xp, g, b):
    return pl.pallas_call(
        _tc_pre_body,
        grid=(_NP // _BR,),
        in_specs=[_row_spec(), _bcast_spec((1, _D)), _bcast_spec((1, _D))],
        out_specs=[_row_spec(), _row_spec()],
        out_shape=[jax.ShapeDtypeStruct((_NP, _D), jnp.float32)] * 2,
    )(xp, g, b)


def _tc_mid(h, num, den, W, b, g, lb, g2, lb2):
    return pl.pallas_call(
        _tc_mid_body,
        grid=(_NP // _BR,),
        in_specs=[_row_spec(), _row_spec(), _row_spec(),
                  _bcast_spec((_D, _D)), _bcast_spec((1, _D)),
                  _bcast_spec((1, _D)), _bcast_spec((1, _D)),
                  _bcast_spec((1, _D)), _bcast_spec((1, _D))],
        out_specs=[_row_spec(), _row_spec(), _row_spec()],
        out_shape=[jax.ShapeDtypeStruct((_NP, _D), jnp.float32)] * 3,
    )(h, num, den, W, b, g, lb, g2, lb2)


def _tc_final(h, num, den, W, b, g, lb, linW, linb, hw, hb, maskf):
    return pl.pallas_call(
        _tc_final_body,
        grid=(_NP // _BR,),
        in_specs=[_row_spec(), _row_spec(), _row_spec(),
                  _bcast_spec((_D, _D)), _bcast_spec((1, _D)),
                  _bcast_spec((1, _D)), _bcast_spec((1, _D)),
                  _bcast_spec((_D, _D)), _bcast_spec((1, _D)),
                  _bcast_spec((1, _D)), _bcast_spec((1, 1)),
                  pl.BlockSpec((_BR, 1), lambda i: (i, 0))],
        out_specs=pl.BlockSpec((_BR, 1), lambda i: (i, 0)),
        out_shape=jax.ShapeDtypeStruct((_NP, 1), jnp.float32),
    )(h, num, den, W, b, g, lb, linW, linb, hw, hb, maskf)


def kernel(x, edge_index, regression_mask, ln_g, ln_b, conv_W, conv_b,
           lin_W, lin_b, head_W, head_b):
    n, d = x.shape
    e = edge_index.shape[1]
    assert (n, d, e, conv_W.shape[0]) == (_N, _D, _E, _L)

    f32 = jnp.float32
    xp = jnp.concatenate([x, jnp.zeros((_NP - _N, _D), f32)], axis=0)
    maskf = jnp.concatenate(
        [regression_mask.astype(f32), jnp.zeros((_NP - _N,), f32)]
    ).reshape(_NP, 1)

    pad = jnp.full((_EP - _E,), _N, jnp.int32)
    src3 = jnp.concatenate([edge_index[0].astype(jnp.int32), pad]
                           ).reshape(_NT, _CH, _CHUNK)
    dst3 = jnp.concatenate([edge_index[1].astype(jnp.int32), pad]
                           ).reshape(_NT, _CH, _CHUNK)

    def r1(v):
        return v.reshape(1, -1)

    h = xp
    w, wp = _tc_pre(xp, r1(ln_g[0]), r1(ln_b[0]))
    for l in range(_L):
        den, num = _sc_segment_sums(w, wp, src3, dst3)
        if l < _L - 1:
            h, w, wp = _tc_mid(h, num, den, conv_W[l], r1(conv_b[l]),
                               r1(ln_g[l]), r1(ln_b[l]),
                               r1(ln_g[l + 1]), r1(ln_b[l + 1]))
        else:
            out2 = _tc_final(h, num, den, conv_W[l], r1(conv_b[l]),
                             r1(ln_g[l]), r1(ln_b[l]),
                             lin_W, r1(lin_b), r1(head_W[:, 0]),
                             head_b.reshape(1, 1), maskf)
    return out2[:_N, 0]


# speedup vs baseline: 2.4753x; 1.0375x over previous
import functools

import jax
import jax.numpy as jnp
from jax import lax
from jax.experimental import pallas as pl
from jax.experimental.pallas import tpu as pltpu
from jax.experimental.pallas import tpu_sc as plsc

_N = 10000
_E = 320000
_D = 128
_L = 6

_NT = 16
_CHUNK = 128
_NP = 10240
_ROWS_PER_TILE = _NP // _NT
_EP = 327680
_CH = _EP // (_NT * _CHUNK)
_G = 16


def _sc_body(w_hbm, wp_hbm, src_hbm, dst_hbm, den_hbm, num_hbm,
             sidx, didx, rows, acc, sem0, sem1):
    cid = lax.axis_index("c")
    tid = lax.axis_index("s")
    sems = (sem0, sem1)

    def run_core(tab_hbm, out_hbm):
        zv = jnp.zeros((16,), jnp.float32)

        def zrow(i, carry):
            for c in range(8):
                rows[0, i, pl.ds(c * 16, 16)] = zv
            return carry

        lax.fori_loop(0, 128, zrow, 0)
        base = tid * _ROWS_PER_TILE
        for k in range(_ROWS_PER_TILE // 128):
            pltpu.sync_copy(rows.at[0], acc.at[pl.ds(base + k * 128, 128)])
        plsc.subcore_barrier()

        def group(g, carry):
            gbase = g * _G
            pltpu.sync_copy(src_hbm.at[tid, pl.ds(gbase, _G)], sidx)
            pltpu.sync_copy(dst_hbm.at[tid, pl.ds(gbase, _G)], didx)
            for b in range(2):
                pltpu.async_copy(tab_hbm.at[sidx.at[b]], rows.at[b], sems[b])

            def chunk(i, carry2):
                j = i * 2
                for b in range(2):
                    jj = j + b
                    pltpu.make_async_copy(
                        tab_hbm.at[sidx.at[jj]], rows.at[b], sems[b]).wait()
                    pltpu.sync_copy(rows.at[b], acc.at[didx.at[jj]], add=True)

                    @pl.when(jj + 2 < _G)
                    def _():
                        pltpu.async_copy(
                            tab_hbm.at[sidx.at[jj + 2]], rows.at[b], sems[b])
                return carry2

            lax.fori_loop(0, _G // 2, chunk, 0)
            return carry

        lax.fori_loop(0, _CH // _G, group, 0)
        plsc.subcore_barrier()

        pltpu.sync_copy(acc.at[pl.ds(base, _ROWS_PER_TILE)],
                        out_hbm.at[pl.ds(base, _ROWS_PER_TILE)])

    @pl.when(cid == 0)
    def _():
        run_core(w_hbm, den_hbm)

    @pl.when(cid == 1)
    def _():
        run_core(wp_hbm, num_hbm)


def _sc_segment_sums(w_tab, wp_tab, src3, dst3):
    mesh = plsc.VectorSubcoreMesh(core_axis_name="c", subcore_axis_name="s")
    fn = functools.partial(
        pl.kernel,
        mesh=mesh,
        out_type=[
            jax.ShapeDtypeStruct((_NP, _D), jnp.float32),
            jax.ShapeDtypeStruct((_NP, _D), jnp.float32),
        ],
        scratch_types=[
            pltpu.VMEM((_G, _CHUNK), jnp.int32),
            pltpu.VMEM((_G, _CHUNK), jnp.int32),
            pltpu.VMEM((2, _CHUNK, _D), jnp.float32),
            pltpu.VMEM_SHARED((_NP, _D), jnp.float32),
            pltpu.SemaphoreType.DMA,
            pltpu.SemaphoreType.DMA,
        ],
    )(_sc_body)
    return fn(w_tab, wp_tab, src3, dst3)


_NW = 2 * _NT
_WE = _EP // _NW
_B = 40


def _move_body(combo_hbm, pos_hbm, out_hbm, cin, pin, cout, v16):
    cid = lax.axis_index("c")
    tid = lax.axis_index("s")
    pltpu.sync_copy(combo_hbm.at[cid, tid], cin)
    pltpu.sync_copy(pos_hbm.at[cid, tid], pin)

    def step(i, carry):
        sl = pl.ds(i * 16, 16)
        plsc.store_scatter(cout, [pin[sl]], cin[sl])
        return carry

    lax.fori_loop(0, _WE // 16, step, 0)
    pltpu.sync_copy(cout, out_hbm.at[cid, tid])


def _sc_partition(combo, pos):
    mesh = plsc.VectorSubcoreMesh(core_axis_name="c", subcore_axis_name="s")
    fn = functools.partial(
        pl.kernel,
        mesh=mesh,
        compiler_params=pltpu.CompilerParams(needs_layout_passes=False),
        out_type=jax.ShapeDtypeStruct((2, _NT, _WE), jnp.int32),
        scratch_types=[
            pltpu.VMEM((_WE,), jnp.int32),
            pltpu.VMEM((_WE,), jnp.int32),
            pltpu.VMEM((_WE,), jnp.int32),
            pltpu.VMEM((16,), jnp.int32),
        ],
    )(_move_body)
    return fn(combo, pos)


def _count_pos(keys, nb):
    nw = 2 * _NT
    b = keys.reshape(nw, _WE // 128, 128)
    onehot = (b[..., None] == jnp.arange(nb, dtype=jnp.int32)
              ).astype(jnp.float32)
    lt = jnp.tril(jnp.ones((128, 128), jnp.float32), k=-1)
    within = jnp.einsum("ij,sbjk->sbik", lt, onehot,
                        preferred_element_type=jnp.float32)
    cntblk = jnp.sum(onehot, axis=2)
    blkpre = jnp.cumsum(cntblk, axis=1) - cntblk
    cnt = jnp.sum(cntblk, axis=1)
    off = jnp.cumsum(cnt, axis=1) - cnt
    base = off[:, None, :] + blkpre
    pos = jnp.sum((within + base[:, :, None, :]) * onehot, axis=-1)
    return pos.astype(jnp.int32).reshape(2, _NT, _WE)


def _partition_edges(src_p, dst_p):
    combo = (src_p * 32768 + dst_p).reshape(2, _NT, _WE)
    key = src_p.reshape(2, _NT, _WE)
    combo = _sc_partition(combo, _count_pos(key >> 8, _B))
    return combo >> 15, combo & 32767


_BR = 2048


def _ln_relu(h, g, b):
    mu = jnp.mean(h, axis=1, keepdims=True)
    var = jnp.mean((h - mu) ** 2, axis=1, keepdims=True)
    z = (h - mu) * lax.rsqrt(var + 1e-5) * g + b
    return jnp.maximum(z, 0.0)


def _tables(z):
    p = z + 1e-7
    w = jnp.exp(p)
    return w, w * p


def _tc_pre_body(x_ref, g_ref, b_ref, w_ref, wp_ref):
    z = _ln_relu(x_ref[...], g_ref[...], b_ref[...])
    w, wp = _tables(z)
    w_ref[...] = w
    wp_ref[...] = wp


def _tc_mid_body(h_ref, num_ref, den_ref, W_ref, b_ref, g_ref, lb_ref,
                 g2_ref, lb2_ref, h2_ref, w_ref, wp_ref):
    h = h_ref[...]
    z = _ln_relu(h, g_ref[...], lb_ref[...])
    aggr = num_ref[...] / jnp.maximum(den_ref[...], 1e-16)
    h2 = h + jnp.dot(z + aggr, W_ref[...],
                     preferred_element_type=jnp.float32) + b_ref[...]
    h2_ref[...] = h2
    z2 = _ln_relu(h2, g2_ref[...], lb2_ref[...])
    w, wp = _tables(z2)
    w_ref[...] = w
    wp_ref[...] = wp


def _tc_final_body(h_ref, num_ref, den_ref, W_ref, b_ref, g_ref, lb_ref,
                   linW_ref, linb_ref, hw_ref, hb_ref, mask_ref, out_ref):
    h = h_ref[...]
    z = _ln_relu(h, g_ref[...], lb_ref[...])
    aggr = num_ref[...] / jnp.maximum(den_ref[...], 1e-16)
    h2 = h + jnp.dot(z + aggr, W_ref[...],
                     preferred_element_type=jnp.float32) + b_ref[...]
    hr = jnp.maximum(
        jnp.dot(h2, linW_ref[...], preferred_element_type=jnp.float32)
        + linb_ref[...], 0.0)
    o = jnp.sum(hr * hw_ref[...], axis=1, keepdims=True) + hb_ref[...]
    out_ref[...] = jnp.where(mask_ref[...] > 0, o, 0.0)


def _row_spec():
    return pl.BlockSpec((_BR, _D), lambda i: (i, 0))


def _bcast_spec(shape):
    return pl.BlockSpec(shape, lambda i: tuple(0 for _ in shape))


def _tc_pre(xp, g, b):
    return pl.pallas_call(
        _tc_pre_body,
        grid=(_NP // _BR,),
        in_specs=[_row_spec(), _bcast_spec((1, _D)), _bcast_spec((1, _D))],
        out_specs=[_row_spec(), _row_spec()],
        out_shape=[jax.ShapeDtypeStruct((_NP, _D), jnp.float32)] * 2,
    )(xp, g, b)


def _tc_mid(h, num, den, W, b, g, lb, g2, lb2):
    return pl.pallas_call(
        _tc_mid_body,
        grid=(_NP // _BR,),
        in_specs=[_row_spec(), _row_spec(), _row_spec(),
                  _bcast_spec((_D, _D)), _bcast_spec((1, _D)),
                  _bcast_spec((1, _D)), _bcast_spec((1, _D)),
                  _bcast_spec((1, _D)), _bcast_spec((1, _D))],
        out_specs=[_row_spec(), _row_spec(), _row_spec()],
        out_shape=[jax.ShapeDtypeStruct((_NP, _D), jnp.float32)] * 3,
    )(h, num, den, W, b, g, lb, g2, lb2)


def _tc_final(h, num, den, W, b, g, lb, linW, linb, hw, hb, maskf):
    return pl.pallas_call(
        _tc_final_body,
        grid=(_NP // _BR,),
        in_specs=[_row_spec(), _row_spec(), _row_spec(),
                  _bcast_spec((_D, _D)), _bcast_spec((1, _D)),
                  _bcast_spec((1, _D)), _bcast_spec((1, _D)),
                  _bcast_spec((_D, _D)), _bcast_spec((1, _D)),
                  _bcast_spec((1, _D)), _bcast_spec((1, 1)),
                  pl.BlockSpec((_BR, 1), lambda i: (i, 0))],
        out_specs=pl.BlockSpec((_BR, 1), lambda i: (i, 0)),
        out_shape=jax.ShapeDtypeStruct((_NP, 1), jnp.float32),
    )(h, num, den, W, b, g, lb, linW, linb, hw, hb, maskf)


def kernel(x, edge_index, regression_mask, ln_g, ln_b, conv_W, conv_b,
           lin_W, lin_b, head_W, head_b):
    n, d = x.shape
    e = edge_index.shape[1]
    assert (n, d, e, conv_W.shape[0]) == (_N, _D, _E, _L)

    f32 = jnp.float32
    xp = jnp.concatenate([x, jnp.zeros((_NP - _N, _D), f32)], axis=0)
    maskf = jnp.concatenate(
        [regression_mask.astype(f32), jnp.zeros((_NP - _N,), f32)]
    ).reshape(_NP, 1)

    pad = jnp.full((_EP - _E,), _N, jnp.int32)
    src_p = jnp.concatenate([edge_index[0].astype(jnp.int32), pad])
    dst_p = jnp.concatenate([edge_index[1].astype(jnp.int32), pad])
    src_m, dst_m = _partition_edges(src_p, dst_p)
    src3 = src_m.reshape(_NT, _CH, _CHUNK)
    dst3 = dst_m.reshape(_NT, _CH, _CHUNK)

    def r1(v):
        return v.reshape(1, -1)

    h = xp
    w, wp = _tc_pre(xp, r1(ln_g[0]), r1(ln_b[0]))
    for l in range(_L):
        den, num = _sc_segment_sums(w, wp, src3, dst3)
        if l < _L - 1:
            h, w, wp = _tc_mid(h, num, den, conv_W[l], r1(conv_b[l]),
                               r1(ln_g[l]), r1(ln_b[l]),
                               r1(ln_g[l + 1]), r1(ln_b[l + 1]))
        else:
            out2 = _tc_final(h, num, den, conv_W[l], r1(conv_b[l]),
                             r1(ln_g[l]), r1(ln_b[l]),
                             lin_W, r1(lin_b), r1(head_W[:, 0]),
                             head_b.reshape(1, 1), maskf)
    return out2[:_N, 0]


# speedup vs baseline: 2.4797x; 1.0018x over previous
import functools

import jax
import jax.numpy as jnp
from jax import lax
from jax.experimental import pallas as pl
from jax.experimental.pallas import tpu as pltpu
from jax.experimental.pallas import tpu_sc as plsc

_N = 10000
_E = 320000
_D = 128
_L = 6

_NT = 16
_CHUNK = 128
_NP = 10240
_ROWS_PER_TILE = _NP // _NT
_EP = 327680
_CH = _EP // (_NT * _CHUNK)
_G = 16


def _sc_body(w_hbm, wp_hbm, src_hbm, dst_hbm, den_hbm, num_hbm,
             sidx, didx, rows, acc, sem0, sem1):
    cid = lax.axis_index("c")
    tid = lax.axis_index("s")
    sems = (sem0, sem1)

    def run_core(tab_hbm, out_hbm):
        zv = jnp.zeros((16,), jnp.float32)

        def zrow(i, carry):
            for c in range(8):
                rows[0, i, pl.ds(c * 16, 16)] = zv
            return carry

        lax.fori_loop(0, 128, zrow, 0)
        base = tid * _ROWS_PER_TILE
        for k in range(_ROWS_PER_TILE // 128):
            pltpu.sync_copy(rows.at[0], acc.at[pl.ds(base + k * 128, 128)])
        plsc.subcore_barrier()

        def group(g, carry):
            gbase = g * _G
            pltpu.sync_copy(src_hbm.at[tid, pl.ds(gbase, _G)], sidx)
            pltpu.sync_copy(dst_hbm.at[tid, pl.ds(gbase, _G)], didx)
            for b in range(2):
                pltpu.async_copy(tab_hbm.at[sidx.at[b]], rows.at[b], sems[b])

            def chunk(i, carry2):
                j = i * 2
                for b in range(2):
                    jj = j + b
                    pltpu.make_async_copy(
                        tab_hbm.at[sidx.at[jj]], rows.at[b], sems[b]).wait()
                    pltpu.sync_copy(rows.at[b], acc.at[didx.at[jj]], add=True)

                    @pl.when(jj + 2 < _G)
                    def _():
                        pltpu.async_copy(
                            tab_hbm.at[sidx.at[jj + 2]], rows.at[b], sems[b])
                return carry2

            lax.fori_loop(0, _G // 2, chunk, 0)
            return carry

        lax.fori_loop(0, _CH // _G, group, 0)
        plsc.subcore_barrier()

        pltpu.sync_copy(acc.at[pl.ds(base, _ROWS_PER_TILE)],
                        out_hbm.at[pl.ds(base, _ROWS_PER_TILE)])

    @pl.when(cid == 0)
    def _():
        run_core(w_hbm, den_hbm)

    @pl.when(cid == 1)
    def _():
        run_core(wp_hbm, num_hbm)


def _sc_segment_sums(w_tab, wp_tab, src3, dst3):
    mesh = plsc.VectorSubcoreMesh(core_axis_name="c", subcore_axis_name="s")
    fn = functools.partial(
        pl.kernel,
        mesh=mesh,
        out_type=[
            jax.ShapeDtypeStruct((_NP, _D), jnp.float32),
            jax.ShapeDtypeStruct((_NP, _D), jnp.float32),
        ],
        scratch_types=[
            pltpu.VMEM((_G, _CHUNK), jnp.int32),
            pltpu.VMEM((_G, _CHUNK), jnp.int32),
            pltpu.VMEM((2, _CHUNK, _D), jnp.float32),
            pltpu.VMEM_SHARED((_NP, _D), jnp.float32),
            pltpu.SemaphoreType.DMA,
            pltpu.SemaphoreType.DMA,
        ],
    )(_sc_body)
    return fn(w_tab, wp_tab, src3, dst3)


_NW = 2 * _NT
_WE = _EP // _NW
_B = 40


def _move_body(combo_hbm, pos_hbm, out_hbm, cin, pin, cout):
    cid = lax.axis_index("c")
    tid = lax.axis_index("s")
    pltpu.sync_copy(combo_hbm.at[cid, tid], cin)
    pltpu.sync_copy(pos_hbm.at[cid, tid], pin)

    def step(i, carry):
        sl = pl.ds(i * 16, 16)
        plsc.store_scatter(cout, [pin[sl]], cin[sl])
        return carry

    lax.fori_loop(0, _WE // 16, step, 0)
    pltpu.sync_copy(cout, out_hbm.at[cid, tid])


def _sc_partition(combo, pos):
    mesh = plsc.VectorSubcoreMesh(core_axis_name="c", subcore_axis_name="s")
    fn = functools.partial(
        pl.kernel,
        mesh=mesh,
        compiler_params=pltpu.CompilerParams(needs_layout_passes=False),
        out_type=jax.ShapeDtypeStruct((2, _NT, _WE), jnp.int32),
        scratch_types=[
            pltpu.VMEM((_WE,), jnp.int32),
            pltpu.VMEM((_WE,), jnp.int32),
            pltpu.VMEM((_WE,), jnp.int32),
        ],
    )(_move_body)
    return fn(combo, pos)


def _count_pos(keys, nb):
    nw = 2 * _NT
    b = keys.reshape(nw, _WE // 128, 128)
    onehot = (b[..., None] == jnp.arange(nb, dtype=jnp.int32)
              ).astype(jnp.float32)
    lt = jnp.tril(jnp.ones((128, 128), jnp.float32), k=-1)
    within = jnp.einsum("ij,sbjk->sbik", lt, onehot,
                        preferred_element_type=jnp.float32)
    cntblk = jnp.sum(onehot, axis=2)
    blkpre = jnp.cumsum(cntblk, axis=1) - cntblk
    cnt = jnp.sum(cntblk, axis=1)
    off = jnp.cumsum(cnt, axis=1) - cnt
    base = off[:, None, :] + blkpre
    pos = jnp.sum((within + base[:, :, None, :]) * onehot, axis=-1)
    return pos.astype(jnp.int32).reshape(2, _NT, _WE)


def _partition_edges(src_p, dst_p):
    combo = (src_p * 32768 + dst_p).reshape(2, _NT, _WE)
    key = src_p.reshape(2, _NT, _WE)
    combo = _sc_partition(combo, _count_pos(key >> 8, _B))
    return combo >> 15, combo & 32767


_BR = 2048


def _ln_relu(h, g, b):
    mu = jnp.mean(h, axis=1, keepdims=True)
    var = jnp.mean((h - mu) ** 2, axis=1, keepdims=True)
    z = (h - mu) * lax.rsqrt(var + 1e-5) * g + b
    return jnp.maximum(z, 0.0)


def _tables(z):
    p = z + 1e-7
    w = jnp.exp(p)
    return w, w * p


def _tc_pre_body(x_ref, g_ref, b_ref, w_ref, wp_ref):
    z = _ln_relu(x_ref[...], g_ref[...], b_ref[...])
    w, wp = _tables(z)
    w_ref[...] = w
    wp_ref[...] = wp


def _tc_mid_body(h_ref, num_ref, den_ref, W_ref, b_ref, g_ref, lb_ref,
                 g2_ref, lb2_ref, h2_ref, w_ref, wp_ref):
    h = h_ref[...]
    z = _ln_relu(h, g_ref[...], lb_ref[...])
    aggr = num_ref[...] / jnp.maximum(den_ref[...], 1e-16)
    h2 = h + jnp.dot(z + aggr, W_ref[...],
                     preferred_element_type=jnp.float32) + b_ref[...]
    h2_ref[...] = h2
    z2 = _ln_relu(h2, g2_ref[...], lb2_ref[...])
    w, wp = _tables(z2)
    w_ref[...] = w
    wp_ref[...] = wp


def _tc_final_body(h_ref, num_ref, den_ref, W_ref, b_ref, g_ref, lb_ref,
                   linW_ref, linb_ref, hw_ref, hb_ref, mask_ref, out_ref):
    h = h_ref[...]
    z = _ln_relu(h, g_ref[...], lb_ref[...])
    aggr = num_ref[...] / jnp.maximum(den_ref[...], 1e-16)
    h2 = h + jnp.dot(z + aggr, W_ref[...],
                     preferred_element_type=jnp.float32) + b_ref[...]
    hr = jnp.maximum(
        jnp.dot(h2, linW_ref[...], preferred_element_type=jnp.float32)
        + linb_ref[...], 0.0)
    o = jnp.sum(hr * hw_ref[...], axis=1, keepdims=True) + hb_ref[...]
    out_ref[...] = jnp.where(mask_ref[...] > 0, o, 0.0)


def _row_spec():
    return pl.BlockSpec((_BR, _D), lambda i: (i, 0))


def _bcast_spec(shape):
    return pl.BlockSpec(shape, lambda i: tuple(0 for _ in shape))


def _tc_pre(xp, g, b):
    return pl.pallas_call(
        _tc_pre_body,
        grid=(_NP // _BR,),
        in_specs=[_row_spec(), _bcast_spec((1, _D)), _bcast_spec((1, _D))],
        out_specs=[_row_spec(), _row_spec()],
        out_shape=[jax.ShapeDtypeStruct((_NP, _D), jnp.float32)] * 2,
    )(xp, g, b)


def _tc_mid(h, num, den, W, b, g, lb, g2, lb2):
    return pl.pallas_call(
        _tc_mid_body,
        grid=(_NP // _BR,),
        in_specs=[_row_spec(), _row_spec(), _row_spec(),
                  _bcast_spec((_D, _D)), _bcast_spec((1, _D)),
                  _bcast_spec((1, _D)), _bcast_spec((1, _D)),
                  _bcast_spec((1, _D)), _bcast_spec((1, _D))],
        out_specs=[_row_spec(), _row_spec(), _row_spec()],
        out_shape=[jax.ShapeDtypeStruct((_NP, _D), jnp.float32)] * 3,
    )(h, num, den, W, b, g, lb, g2, lb2)


def _tc_final(h, num, den, W, b, g, lb, linW, linb, hw, hb, maskf):
    return pl.pallas_call(
        _tc_final_body,
        grid=(_NP // _BR,),
        in_specs=[_row_spec(), _row_spec(), _row_spec(),
                  _bcast_spec((_D, _D)), _bcast_spec((1, _D)),
                  _bcast_spec((1, _D)), _bcast_spec((1, _D)),
                  _bcast_spec((_D, _D)), _bcast_spec((1, _D)),
                  _bcast_spec((1, _D)), _bcast_spec((1, 1)),
                  pl.BlockSpec((_BR, 1), lambda i: (i, 0))],
        out_specs=pl.BlockSpec((_BR, 1), lambda i: (i, 0)),
        out_shape=jax.ShapeDtypeStruct((_NP, 1), jnp.float32),
    )(h, num, den, W, b, g, lb, linW, linb, hw, hb, maskf)


def kernel(x, edge_index, regression_mask, ln_g, ln_b, conv_W, conv_b,
           lin_W, lin_b, head_W, head_b):
    n, d = x.shape
    e = edge_index.shape[1]
    assert (n, d, e, conv_W.shape[0]) == (_N, _D, _E, _L)

    f32 = jnp.float32
    xp = jnp.concatenate([x, jnp.zeros((_NP - _N, _D), f32)], axis=0)
    maskf = jnp.concatenate(
        [regression_mask.astype(f32), jnp.zeros((_NP - _N,), f32)]
    ).reshape(_NP, 1)

    pad = jnp.full((_EP - _E,), _N, jnp.int32)
    src_p = jnp.concatenate([edge_index[0].astype(jnp.int32), pad])
    dst_p = jnp.concatenate([edge_index[1].astype(jnp.int32), pad])
    src_m, dst_m = _partition_edges(src_p, dst_p)
    src3 = src_m.reshape(_NT, _CH, _CHUNK)
    dst3 = dst_m.reshape(_NT, _CH, _CHUNK)

    def r1(v):
        return v.reshape(1, -1)

    h = xp
    w, wp = _tc_pre(xp, r1(ln_g[0]), r1(ln_b[0]))
    for l in range(_L):
        den, num = _sc_segment_sums(w, wp, src3, dst3)
        if l < _L - 1:
            h, w, wp = _tc_mid(h, num, den, conv_W[l], r1(conv_b[l]),
                               r1(ln_g[l]), r1(ln_b[l]),
                               r1(ln_g[l + 1]), r1(ln_b[l + 1]))
        else:
            out2 = _tc_final(h, num, den, conv_W[l], r1(conv_b[l]),
                             r1(ln_g[l]), r1(ln_b[l]),
                             lin_W, r1(lin_b), r1(head_W[:, 0]),
                             head_b.reshape(1, 1), maskf)
    return out2[:_N, 0]
